# Initial kernel scaffold; baseline (speedup 1.0000x reference)
import jax
import jax.numpy as jnp
from jax.experimental import pallas as pl


def kernel(input_ids, weights):
    raise NotImplementedError("write your pallas kernel here")


# speedup vs baseline: 1.8806x; 1.8806x over previous
import functools

import jax
import jax.numpy as jnp
from jax import lax
from jax.experimental import pallas as pl
from jax.experimental.pallas import tpu as pltpu
from jax.experimental.pallas import tpu_sc as plsc

PAD = 1
NC, NS, L = 2, 16, 16
NW = NC * NS

B, S, D = 4, 4096, 768
TOT = B * S
PW = TOT // NW
WPB = S // PW
CHUNK = 64
NCHUNK = PW // CHUNK
VREGS_PER_CHUNK = CHUNK // L


def _body(ids_hbm, w_hbm, out_hbm, row_v, idx_v, g0, g1, gs0, gs1, ss0, ss1):
    wid = lax.axis_index("s") * NC + lax.axis_index("c")
    b = wid // WPB
    c = wid % WPB

    pltpu.sync_copy(ids_hbm.at[pl.ds(b * S, S)], row_v)

    def count_body(j, acc):
        v = row_v[pl.ds(j * L, L)]
        return acc + jnp.sum((v != PAD).astype(jnp.int32))

    carry = lax.fori_loop(0, c * (PW // L), count_body, jnp.int32(0))

    cbase = c * PW
    for i in range(PW // L):
        v = row_v[pl.ds(cbase + i * L, L)]
        mi = (v != PAD).astype(jnp.int32)
        cs = plsc.cumsum(mi)
        pos = (carry + cs) * mi + 1
        idx_v[i // VREGS_PER_CHUNK, pl.ds((i % VREGS_PER_CHUNK) * L, L)] = pos
        carry = carry + jnp.sum(mi)

    obase = wid * PW
    bufs = (g0, g1)
    gsems = (gs0, gs1)
    ssems = (ss0, ss1)

    def gather(ci):
        return pltpu.async_copy(w_hbm.at[idx_v.at[ci]], bufs[ci % 2], gsems[ci % 2])

    def scatter(ci):
        dst = out_hbm.at[pl.ds(obase + ci * CHUNK, CHUNK)]
        return pltpu.async_copy(bufs[ci % 2], dst, ssems[ci % 2])

    gh = [None] * NCHUNK
    sh = [None] * NCHUNK
    gh[0] = gather(0)
    for ci in range(NCHUNK):
        gh[ci].wait()
        sh[ci] = scatter(ci)
        nx = ci + 1
        if nx < NCHUNK:
            if nx >= 2:
                sh[nx - 2].wait()
            gh[nx] = gather(nx)
    sh[NCHUNK - 2].wait()
    sh[NCHUNK - 1].wait()


@jax.jit
def _sc_gather(ids_flat, weights):
    mesh = plsc.VectorSubcoreMesh(
        core_axis_name="c", subcore_axis_name="s", num_cores=NC, num_subcores=NS
    )
    return pl.kernel(
        _body,
        out_type=jax.ShapeDtypeStruct((TOT, D), jnp.float32),
        mesh=mesh,
        compiler_params=pltpu.CompilerParams(needs_layout_passes=False),
        scratch_types=[
            pltpu.VMEM((S,), jnp.int32),
            pltpu.VMEM((NCHUNK, CHUNK), jnp.int32),
            pltpu.VMEM((CHUNK, D), jnp.float32),
            pltpu.VMEM((CHUNK, D), jnp.float32),
            pltpu.SemaphoreType.DMA,
            pltpu.SemaphoreType.DMA,
            pltpu.SemaphoreType.DMA,
            pltpu.SemaphoreType.DMA,
        ],
    )(ids_flat, weights)


def kernel(input_ids, weights):
    assert input_ids.shape == (B, S)
    assert weights.shape[1] == D
    out = _sc_gather(input_ids.reshape(-1), weights)
    return out.reshape(B, S, D)


# speedup vs baseline: 1.9337x; 1.0282x over previous
import functools

import jax
import jax.numpy as jnp
from jax import lax
from jax.experimental import pallas as pl
from jax.experimental.pallas import tpu as pltpu
from jax.experimental.pallas import tpu_sc as plsc

PAD = 1
NC, NS, L = 2, 16, 16
NW = NC * NS

B, S, D = 4, 4096, 768
TOT = B * S
PW = TOT // NW
WPB = S // PW
CHUNK = 64
NCHUNK = PW // CHUNK
VPC = CHUNK // L


def _body(ids_hbm, w_hbm, out_hbm, row_v, idx_v, g0, g1, gs0, gs1, ss0, ss1):
    wid = lax.axis_index("s") * NC + lax.axis_index("c")
    b = wid // WPB
    c = wid % WPB

    pltpu.sync_copy(ids_hbm.at[pl.ds(b * S, S)], row_v)

    def count_body(j, acc):
        v = row_v[pl.ds(j * L, L)]
        return acc + jnp.where(v == PAD, 0, 1)

    acc = lax.fori_loop(0, c * (PW // L), count_body, jnp.zeros((L,), jnp.int32))
    carry = jnp.sum(acc)

    obase = wid * PW
    bufs = (g0, g1)
    gsems = (gs0, gs1)
    ssems = (ss0, ss1)

    def gather(ci):
        return pltpu.async_copy(w_hbm.at[idx_v.at[ci]], bufs[ci % 2], gsems[ci % 2])

    def scatter(ci):
        dst = out_hbm.at[pl.ds(obase + ci * CHUNK, CHUNK)]
        return pltpu.async_copy(bufs[ci % 2], dst, ssems[ci % 2])

    cbase = c * PW
    gh = [None] * NCHUNK
    sh = [None] * NCHUNK
    for ci in range(NCHUNK):
        for k in range(VPC):
            v = row_v[pl.ds(cbase + (ci * VPC + k) * L, L)]
            mi = jnp.where(v == PAD, 0, 1)
            cs = plsc.cumsum(mi)
            idx_v[ci, pl.ds(k * L, L)] = (carry + cs) * mi + 1
            carry = carry + jnp.sum(mi)
        if ci >= 2:
            sh[ci - 2].wait()
        gh[ci] = gather(ci)
        if ci >= 1:
            gh[ci - 1].wait()
            sh[ci - 1] = scatter(ci - 1)
    gh[NCHUNK - 1].wait()
    sh[NCHUNK - 1] = scatter(NCHUNK - 1)
    sh[NCHUNK - 2].wait()
    sh[NCHUNK - 1].wait()


@jax.jit
def _sc_gather(ids_flat, weights):
    mesh = plsc.VectorSubcoreMesh(
        core_axis_name="c", subcore_axis_name="s", num_cores=NC, num_subcores=NS
    )
    return pl.kernel(
        _body,
        out_type=jax.ShapeDtypeStruct((TOT, D), jnp.float32),
        mesh=mesh,
        compiler_params=pltpu.CompilerParams(needs_layout_passes=False),
        scratch_types=[
            pltpu.VMEM((S,), jnp.int32),
            pltpu.VMEM((NCHUNK, CHUNK), jnp.int32),
            pltpu.VMEM((CHUNK, D), jnp.float32),
            pltpu.VMEM((CHUNK, D), jnp.float32),
            pltpu.SemaphoreType.DMA,
            pltpu.SemaphoreType.DMA,
            pltpu.SemaphoreType.DMA,
            pltpu.SemaphoreType.DMA,
        ],
    )(ids_flat, weights)


def kernel(input_ids, weights):
    assert input_ids.shape == (B, S)
    assert weights.shape[1] == D
    out = _sc_gather(input_ids.reshape(-1), weights)
    return out.reshape(B, S, D)


# speedup vs baseline: 1.9357x; 1.0010x over previous
import functools

import jax
import jax.numpy as jnp
from jax import lax
from jax.experimental import pallas as pl
from jax.experimental.pallas import tpu as pltpu
from jax.experimental.pallas import tpu_sc as plsc

PAD = 1
NC, NS, L = 2, 16, 16
NW = NC * NS

B, S, D = 4, 4096, 768
TOT = B * S
PW = TOT // NW
WPB = S // PW
CHUNK = 64
NCHUNK = PW // CHUNK
VPC = CHUNK // L


def _body(ids_hbm, w_hbm, out_hbm, row_v, idx_v, g0, g1, gs0, gs1, ss0, ss1):
    wid = lax.axis_index("s") * NC + lax.axis_index("c")
    b = wid // WPB
    c = wid % WPB

    pltpu.sync_copy(ids_hbm.at[pl.ds(b * S, S)], row_v)

    def count_body(j, acc):
        v = row_v[pl.ds(j * L, L)]
        return acc + jnp.where(v == PAD, 0, 1)

    acc = lax.fori_loop(0, c * (PW // L), count_body, jnp.zeros((L,), jnp.int32))
    carry = jnp.sum(acc)

    obase = wid * PW
    bufs = (g0, g1)
    gsems = (gs0, gs1)
    ssems = (ss0, ss1)

    def gather(ci):
        return pltpu.async_copy(w_hbm.at[idx_v.at[ci]], bufs[ci % 2], gsems[ci % 2])

    def scatter(ci):
        dst = out_hbm.at[pl.ds(obase + ci * CHUNK, CHUNK)]
        return pltpu.async_copy(bufs[ci % 2], dst, ssems[ci % 2])

    cbase = c * PW
    gh = [None] * NCHUNK
    sh = [None] * NCHUNK
    for ci in range(NCHUNK):
        for k in range(VPC):
            v = row_v[pl.ds(cbase + (ci * VPC + k) * L, L)]
            mi = jnp.where(v == PAD, 0, 1)
            cs = plsc.cumsum(mi)
            idx_v[ci, pl.ds(k * L, L)] = (carry + cs) * mi + 1
            carry = carry + jnp.sum(mi)
        if ci >= 2:
            sh[ci - 2].wait()
        gh[ci] = gather(ci)
        if ci >= 1:
            gh[ci - 1].wait()
            sh[ci - 1] = scatter(ci - 1)
    gh[NCHUNK - 1].wait()
    sh[NCHUNK - 1] = scatter(NCHUNK - 1)
    sh[NCHUNK - 2].wait()
    sh[NCHUNK - 1].wait()


@jax.jit
def _sc_gather(ids_flat, weights):
    mesh = plsc.VectorSubcoreMesh(
        core_axis_name="c", subcore_axis_name="s", num_cores=NC, num_subcores=NS
    )
    return pl.kernel(
        _body,
        out_type=jax.ShapeDtypeStruct((TOT, D), jnp.float32),
        mesh=mesh,
        compiler_params=pltpu.CompilerParams(
            needs_layout_passes=False,
            disable_bounds_checks=True,
            disable_semaphore_checks=True,
        ),
        scratch_types=[
            pltpu.VMEM((S,), jnp.int32),
            pltpu.VMEM((NCHUNK, CHUNK), jnp.int32),
            pltpu.VMEM((CHUNK, D), jnp.float32),
            pltpu.VMEM((CHUNK, D), jnp.float32),
            pltpu.SemaphoreType.DMA,
            pltpu.SemaphoreType.DMA,
            pltpu.SemaphoreType.DMA,
            pltpu.SemaphoreType.DMA,
        ],
    )(ids_flat, weights)


def kernel(input_ids, weights):
    assert input_ids.shape == (B, S)
    assert weights.shape[1] == D
    out = _sc_gather(input_ids.reshape(-1), weights)
    return out.reshape(B, S, D)
